# baseline (speedup 1.0000x reference)
import functools

import jax
import jax.numpy as jnp
from jax import lax
from jax.experimental import pallas as pl
from jax.experimental.pallas import tpu as pltpu
from jax.experimental.pallas import tpu_sc as plsc

SCALE = 8.0
NC = 2
NS = 16
NW = NC * NS
CR = 8
NBUF = 2
SP = 24
LP = 128


@functools.lru_cache(maxsize=None)
def _make_emb(R, S, V, D):
    rpw = R // NW
    nchunk = rpw // CR
    mesh = plsc.VectorSubcoreMesh(core_axis_name="c", subcore_axis_name="s")

    @functools.partial(
        pl.kernel,
        mesh=mesh,
        compiler_params=pltpu.CompilerParams(use_tc_tiling_on_sc=False),
        out_type=jax.ShapeDtypeStruct((R, SP, LP), jnp.float32),
        scratch_types=[
            pltpu.VMEM((NBUF, CR, SP), jnp.int32),
            pltpu.VMEM((NBUF, CR, SP, D), jnp.float32),
            pltpu.VMEM((NBUF, CR, SP, LP), jnp.float32),
        ]
        + [pltpu.SemaphoreType.DMA for _ in range(2 * NBUF)],
    )
    def emb(idx_hbm, table_hbm, out_hbm, page_v, gbuf_v, sbuf_v, *sems):
        gsems = sems[:NBUF]
        ssems = sems[NBUF:]
        wid = lax.axis_index("s") * NC + lax.axis_index("c")
        row0 = pl.multiple_of(wid * rpw, rpw)

        def fire(c):
            b = c % NBUF
            pltpu.sync_copy(
                idx_hbm.at[pl.ds(row0 + c * CR, CR), pl.ds(0, SP)],
                page_v.at[b])
            return [
                pltpu.async_copy(
                    table_hbm.at[page_v.at[b, r]],
                    gbuf_v.at[b, r], gsems[b])
                for r in range(CR)
            ]

        def scale(b):
            def scale_body(i, carry):
                r = i // S
                s = i % S
                for j in range(D // 16):
                    sbuf_v[b, r, s, pl.ds(j * 16, 16)] = (
                        gbuf_v[b, r, s, pl.ds(j * 16, 16)] * SCALE)
                return carry
            lax.fori_loop(0, CR * S, scale_body, 0)

        ghandles = {}
        shandles = {}
        ghandles[0] = fire(0)
        for c in range(nchunk):
            b = c % NBUF
            n = c + 1
            if n < nchunk:
                if n >= NBUF:
                    shandles.pop(n - NBUF).wait()
                ghandles[n] = fire(n)
            for h in ghandles.pop(c):
                h.wait()
            scale(b)
            roff = row0 + c * CR
            shandles[c] = pltpu.async_copy(
                sbuf_v.at[b], out_hbm.at[pl.ds(roff, CR)], ssems[b])
        for c in sorted(shandles):
            shandles.pop(c).wait()

    return emb


def kernel(X, table):
    R, S = X.shape
    V, D = table.shape
    xp = jnp.pad(X.astype(jnp.int32), ((0, 0), (0, 128 - S)))
    padded = _make_emb(R, S, V, D)(xp, table)
    return lax.slice(padded, (0, 0, 0), (R, S, D))


# speedup vs baseline: 2.4342x; 2.4342x over previous
import functools

import jax
import jax.numpy as jnp
from jax import lax
from jax.experimental import pallas as pl
from jax.experimental.pallas import tpu as pltpu
from jax.experimental.pallas import tpu_sc as plsc

SCALE = 8.0
NC = 2
NS = 16
NW = NC * NS
CR = 8
NBUF = 2
SP = 24
LP = 128


@functools.lru_cache(maxsize=None)
def _make_emb(R, S, V, D):
    rpw = R // NW
    nchunk = rpw // CR
    mesh = plsc.VectorSubcoreMesh(core_axis_name="c", subcore_axis_name="s")

    @functools.partial(
        pl.kernel,
        mesh=mesh,
        compiler_params=pltpu.CompilerParams(use_tc_tiling_on_sc=False),
        out_type=jax.ShapeDtypeStruct((R, SP, LP), jnp.float32),
        scratch_types=[
            pltpu.VMEM((rpw, S), jnp.int32),
            pltpu.VMEM((NBUF, CR, S, D), jnp.float32),
            pltpu.VMEM((NBUF, CR, SP, LP), jnp.float32),
        ]
        + [pltpu.SemaphoreType.DMA for _ in range(2 * NBUF)],
    )
    def emb(idx_hbm, table_hbm, out_hbm, idx_v, gbuf_v, sbuf_v, *sems):
        gsems = sems[:NBUF]
        ssems = sems[NBUF:]
        wid = lax.axis_index("s") * NC + lax.axis_index("c")
        row0 = pl.multiple_of(wid * rpw, rpw)

        pltpu.sync_copy(idx_hbm.at[pl.ds(row0, rpw)], idx_v)

        def fire(c):
            b = c % NBUF
            return [
                pltpu.async_copy(
                    table_hbm.at[idx_v.at[c * CR + r]],
                    gbuf_v.at[b, r], gsems[b])
                for r in range(CR)
            ]

        def scale(b):
            def scale_body(i, carry):
                r = i // S
                s = i % S
                for j in range(D // 16):
                    sbuf_v[b, r, s, pl.ds(j * 16, 16)] = (
                        gbuf_v[b, r, s, pl.ds(j * 16, 16)] * SCALE)
                return carry
            lax.fori_loop(0, CR * S, scale_body, 0)

        ghandles = {}
        shandles = {}
        ghandles[0] = fire(0)
        for c in range(nchunk):
            b = c % NBUF
            n = c + 1
            if n < nchunk:
                if n >= NBUF:
                    shandles.pop(n - NBUF).wait()
                ghandles[n] = fire(n)
            for h in ghandles.pop(c):
                h.wait()
            scale(b)
            roff = row0 + c * CR
            shandles[c] = pltpu.async_copy(
                sbuf_v.at[b], out_hbm.at[pl.ds(roff, CR)], ssems[b])
        for c in sorted(shandles):
            shandles.pop(c).wait()

    return emb


def kernel(X, table):
    R, S = X.shape
    V, D = table.shape
    padded = _make_emb(R, S, V, D)(X.astype(jnp.int32), table)
    return lax.slice(padded, (0, 0, 0), (R, S, D))


# speedup vs baseline: 2.4473x; 1.0054x over previous
import functools

import jax
import jax.numpy as jnp
from jax import lax
from jax.experimental import pallas as pl
from jax.experimental.pallas import tpu as pltpu
from jax.experimental.pallas import tpu_sc as plsc

SCALE = 8.0
NC = 2
NS = 16
NW = NC * NS
CR = 8
NBUF = 2
SP = 24
LP = 128
PAGE = SP * LP
VALID = 20 * LP


@functools.lru_cache(maxsize=None)
def _make_emb(R, S, V, D):
    rpw = R // NW
    nchunk = rpw // CR
    mesh = plsc.VectorSubcoreMesh(core_axis_name="c", subcore_axis_name="s")

    @functools.partial(
        pl.kernel,
        mesh=mesh,
        compiler_params=pltpu.CompilerParams(use_tc_tiling_on_sc=False),
        out_type=jax.ShapeDtypeStruct((R * PAGE,), jnp.float32),
        scratch_types=[
            pltpu.VMEM((rpw, S), jnp.int32),
            pltpu.VMEM((NBUF, CR, S, D), jnp.float32),
            pltpu.VMEM((NBUF, CR * VALID), jnp.float32),
        ]
        + [pltpu.SemaphoreType.DMA for _ in range(2 * NBUF)],
    )
    def emb(idx_hbm, table_hbm, out_hbm, idx_v, gbuf_v, sbuf_v, *sems):
        gsems = sems[:NBUF]
        ssems = sems[NBUF:]
        wid = lax.axis_index("s") * NC + lax.axis_index("c")
        row0 = pl.multiple_of(wid * rpw, rpw)

        pltpu.sync_copy(idx_hbm.at[pl.ds(row0, rpw)], idx_v)

        def fire(c):
            b = c % NBUF
            return [
                pltpu.async_copy(
                    table_hbm.at[idx_v.at[c * CR + r]],
                    gbuf_v.at[b, r], gsems[b])
                for r in range(CR)
            ]

        def scale(b):
            def scale_body(i, carry):
                r = i // S
                s = i % S
                for j in range(D // 16):
                    dst = pl.multiple_of(r * VALID + s * LP + j * 16, 16)
                    sbuf_v[b, pl.ds(dst, 16)] = (
                        gbuf_v[b, r, s, pl.ds(j * 16, 16)] * SCALE)
                return carry
            lax.fori_loop(0, CR * S, scale_body, 0)

        ghandles = {}
        shandles = {}
        ghandles[0] = fire(0)
        for c in range(nchunk):
            b = c % NBUF
            n = c + 1
            if n < nchunk:
                if n >= NBUF:
                    shandles.pop(n - NBUF)
                    pltpu.make_async_copy(
                        out_hbm.at[pl.ds(0, CR * VALID)], sbuf_v.at[(n - NBUF) % NBUF],
                        ssems[(n - NBUF) % NBUF]).wait()
                ghandles[n] = fire(n)
            for h in ghandles.pop(c):
                h.wait()
            scale(b)
            shandles[c] = [
                pltpu.async_copy(
                    sbuf_v.at[b, pl.ds(r * VALID, VALID)],
                    out_hbm.at[pl.ds((row0 + c * CR + r) * PAGE, VALID)],
                    ssems[b])
                for r in range(CR)
            ]
        for c in sorted(shandles):
            shandles.pop(c)
            pltpu.make_async_copy(
                out_hbm.at[pl.ds(0, CR * VALID)], sbuf_v.at[c % NBUF],
                ssems[c % NBUF]).wait()

    return emb


def kernel(X, table):
    R, S = X.shape
    V, D = table.shape
    flat = _make_emb(R, S, V, D)(X.astype(jnp.int32), table)
    padded = flat.reshape(R, SP, LP)
    return lax.slice(padded, (0, 0, 0), (R, S, D))


# speedup vs baseline: 2.6220x; 1.0714x over previous
import functools

import jax
import jax.numpy as jnp
from jax import lax
from jax.experimental import pallas as pl
from jax.experimental.pallas import tpu as pltpu
from jax.experimental.pallas import tpu_sc as plsc

SCALE = 8.0
NC = 2
NS = 16
NW = NC * NS
CR = 8
NBUF = 2
SP = 24
LP = 128
PAGE = SP * LP
VALID = 20 * LP


@functools.lru_cache(maxsize=None)
def _make_emb(R, S, V, D):
    rpw = R // NW
    nchunk = rpw // CR
    npair = nchunk // NBUF
    mesh = plsc.VectorSubcoreMesh(core_axis_name="c", subcore_axis_name="s")

    @functools.partial(
        pl.kernel,
        mesh=mesh,
        compiler_params=pltpu.CompilerParams(use_tc_tiling_on_sc=False),
        out_type=jax.ShapeDtypeStruct((R * PAGE,), jnp.float32),
        scratch_types=[
            pltpu.VMEM((rpw, S), jnp.int32),
            pltpu.VMEM((NBUF, CR, S, D), jnp.float32),
            pltpu.VMEM((NBUF, CR * VALID), jnp.float32),
        ]
        + [pltpu.SemaphoreType.DMA for _ in range(2 * NBUF)],
    )
    def emb(idx_hbm, table_hbm, out_hbm, idx_v, gbuf_v, sbuf_v, *sems):
        gsems = sems[:NBUF]
        ssems = sems[NBUF:]
        wid = lax.axis_index("s") * NC + lax.axis_index("c")
        row0 = pl.multiple_of(wid * rpw, rpw)

        pltpu.sync_copy(idx_hbm.at[pl.ds(row0, rpw)], idx_v)

        def fire(c, b):
            for r in range(CR):
                pltpu.async_copy(table_hbm.at[idx_v.at[c * CR + r]],
                                 gbuf_v.at[b, r], gsems[b])

        def drain_gathers(b):
            pltpu.make_async_copy(
                out_hbm.at[pl.ds(0, CR * S * D)],
                sbuf_v.at[b, pl.ds(0, CR * S * D)], gsems[b]).wait()

        def drain_stores(b):
            pltpu.make_async_copy(
                out_hbm.at[pl.ds(0, CR * VALID)], sbuf_v.at[b],
                ssems[b]).wait()

        def scale(b):
            def scale_body(i, carry):
                r = i // (S // 4)
                s4 = (i % (S // 4)) * 4
                for u in range(4):
                    for j in range(D // 16):
                        dst = pl.multiple_of(
                            r * VALID + (s4 + u) * LP + j * 16, 16)
                        sbuf_v[b, pl.ds(dst, 16)] = (
                            gbuf_v[b, r, s4 + u, pl.ds(j * 16, 16)] * SCALE)
                return carry
            lax.fori_loop(0, CR * (S // 4), scale_body, 0)

        def store(c, b):
            for r in range(CR):
                off = pl.multiple_of((row0 + c * CR + r) * PAGE, 1024)
                pltpu.async_copy(sbuf_v.at[b, pl.ds(r * VALID, VALID)],
                                 out_hbm.at[pl.ds(off, VALID)], ssems[b])

        fire(0, 0)

        def pair_body(co, carry):
            for b in range(NBUF):
                c = co * NBUF + b
                n = c + 1
                nb = (b + 1) % NBUF

                @pl.when(n < nchunk)
                def _prefetch():
                    fire(n, nb)

                drain_gathers(b)

                @pl.when(c >= NBUF)
                def _drain_prev_store():
                    drain_stores(b)

                scale(b)
                store(c, b)
            return carry

        lax.fori_loop(0, npair, pair_body, 0)
        for b in range(NBUF):
            drain_stores(b)

    return emb


def kernel(X, table):
    R, S = X.shape
    V, D = table.shape
    flat = _make_emb(R, S, V, D)(X.astype(jnp.int32), table)
    padded = flat.reshape(R, SP, LP)
    return lax.slice(padded, (0, 0, 0), (R, S, D))
